# baseline (speedup 1.0000x reference)
import functools

import jax
import jax.numpy as jnp
from jax import lax
from jax.experimental import pallas as pl
from jax.experimental.pallas import tpu as pltpu
from jax.experimental.pallas import tpu_sc as plsc

N = 50000
E = 800000
D = 64
HID = 128
K = 3
ALPHA = 0.1

NC = 2
NS = 16
L = 16

HALF = N // NC
ACC_ROWS = 25216
DUMP_BASE = 25088
DUMP_MASK = 127
ZCHUNKS = ACC_ROWS // 128

CE = 384
NB = CE // 128
EPT = E // NS
NFULL = EPT // CE
TAIL = EPT - NFULL * CE

_MESH = plsc.VectorSubcoreMesh(
    core_axis_name="c", subcore_axis_name="s", num_cores=NC, num_subcores=NS)


_DEPT = 25008
_DFULL = 12
_DCE = 2048
_DTAIL_A = _DEPT - _DFULL * _DCE
_DTAIL_B = (E - 31 * _DEPT) - _DFULL * _DCE


def _deg_body(src_hbm, dst_hbm, part_hbm, degv, sb, db, sem):
    c = lax.axis_index("c")
    s = lax.axis_index("s")
    w = c * NS + s
    zeros16 = jnp.zeros((L,), jnp.float32)
    ones16 = jnp.ones((L,), jnp.float32)

    @pl.loop(0, 625)
    def _zero(i):
        base = i * 160
        for v in range(10):
            degv[pl.ds(base + v * L, L)] = zeros16

    start = w * _DEPT

    def do_chunk(off, ce):
        pltpu.sync_copy(src_hbm.at[pl.ds(off, ce)], sb.at[pl.ds(0, ce)])
        pltpu.sync_copy(dst_hbm.at[pl.ds(off, ce)], db.at[pl.ds(0, ce)])
        for j in range(ce // L):
            siv = sb[pl.ds(j * L, L)]
            plsc.addupdate_scatter(degv, [siv], ones16)
            div = db[pl.ds(j * L, L)] + N
            plsc.addupdate_scatter(degv, [div], ones16)

    @pl.loop(0, _DFULL)
    def _main(ci):
        do_chunk(start + ci * _DCE, _DCE)

    @pl.when(w < 31)
    def _tail_a():
        do_chunk(start + _DFULL * _DCE, _DTAIL_A)

    @pl.when(w == 31)
    def _tail_b():
        do_chunk(start + _DFULL * _DCE, _DTAIL_B)

    pltpu.sync_copy(degv, part_hbm.at[w])


_deg_kernel = functools.partial(
    pl.kernel,
    out_type=jax.ShapeDtypeStruct((NC * NS, 2 * N), jnp.float32),
    mesh=_MESH,
    scratch_types=[
        pltpu.VMEM((2 * N,), jnp.float32),
        pltpu.VMEM((_DCE,), jnp.int32),
        pltpu.VMEM((_DCE,), jnp.int32),
        pltpu.SemaphoreType.DMA,
    ],
    compiler_params=pltpu.CompilerParams(needs_layout_passes=False),
)(_deg_body)


def _seg_body(msrc_hbm, src_hbm, dst_hbm, zrows_hbm, out_hbm,
              acc, sb, db, sflat, dflat, s2d, d2d, rows, sem):
    c = lax.axis_index("c")
    s = lax.axis_index("s")
    lo = c * HALF
    iota = lax.iota(jnp.int32, L)
    zeros16i = jnp.zeros((L,), jnp.int32)

    for i in range(13):
        ch = s + NS * i

        @pl.when(ch < ZCHUNKS)
        def _z():
            pltpu.sync_copy(zrows_hbm, acc.at[pl.ds(ch * 128, 128), :])

    for v in range(CE // L):
        sflat[pl.ds(v * L, L)] = zeros16i
        dflat[pl.ds(v * L, L)] = DUMP_BASE + ((v * L + iota) & DUMP_MASK)

    plsc.subcore_barrier()

    start = s * EPT

    def do_chunk(off, ce):
        nv = ce // L
        pltpu.sync_copy(src_hbm.at[pl.ds(off, ce)], sb.at[pl.ds(0, ce)])
        pltpu.sync_copy(dst_hbm.at[pl.ds(off, ce)], db.at[pl.ds(0, ce)])
        m = jnp.int32(0)
        for j in range(nv):
            dv = db[pl.ds(j * L, L)]
            sv = sb[pl.ds(j * L, L)]
            loc = dv - lo
            inr = (loc >= 0) & (loc < HALF)
            plsc.store_compressed(dflat.at[pl.ds(m, L)], loc, mask=inr)
            plsc.store_compressed(sflat.at[pl.ds(m, L)], sv, mask=inr)
            m = m + plsc.all_reduce_population_count(inr)[0]
        nb = (m + 127) >> 7
        end = nb << 7
        for i in range(8):
            pos = m + L * i

            @pl.when(pos < end)
            def _fill():
                dflat[pl.ds(pos, L)] = DUMP_BASE + ((pos + iota) & DUMP_MASK)

        for b in range(NB):

            @pl.when(b < nb)
            def _re():
                for v in range(128 // L):
                    s2d[b, pl.ds(v * L, L)] = sflat[pl.ds(b * 128 + v * L, L)]
                    d2d[b, pl.ds(v * L, L)] = dflat[pl.ds(b * 128 + v * L, L)]

        for b in range(NB):

            @pl.when(b < nb)
            def _g():
                pltpu.async_copy(msrc_hbm.at[s2d.at[b]],
                                 rows.at[pl.ds(b * 128, 128)], sem)

        for b in range(NB):

            @pl.when(b < nb)
            def _gw():
                pltpu.make_async_copy(msrc_hbm.at[s2d.at[b]],
                                      rows.at[pl.ds(b * 128, 128)], sem).wait()

        for b in range(NB):

            @pl.when(b < nb)
            def _sc():
                pltpu.sync_copy(rows.at[pl.ds(b * 128, 128)],
                                acc.at[d2d.at[b]], add=True)

    @pl.loop(0, NFULL)
    def _main(ci):
        do_chunk(start + ci * CE, CE)

    do_chunk(start + NFULL * CE, TAIL)

    plsc.subcore_barrier()

    @pl.when(s < 15)
    def _cp():
        pltpu.sync_copy(acc.at[pl.ds(s * 1568, 1568), :],
                        out_hbm.at[pl.ds(lo + s * 1568, 1568), :])

    @pl.when(s == 15)
    def _cp_last():
        pltpu.sync_copy(acc.at[pl.ds(15 * 1568, 1480), :],
                        out_hbm.at[pl.ds(lo + 15 * 1568, 1480), :])


_seg_kernel = functools.partial(
    pl.kernel,
    out_type=jax.ShapeDtypeStruct((N, D), jnp.float32),
    mesh=_MESH,
    scratch_types=[
        pltpu.VMEM_SHARED((ACC_ROWS, D), jnp.float32),
        pltpu.VMEM((CE,), jnp.int32),
        pltpu.VMEM((CE,), jnp.int32),
        pltpu.VMEM((CE + L,), jnp.int32),
        pltpu.VMEM((CE + L,), jnp.int32),
        pltpu.VMEM((NB, 128), jnp.int32),
        pltpu.VMEM((NB, 128), jnp.int32),
        pltpu.VMEM((CE, D), jnp.float32),
        pltpu.SemaphoreType.DMA,
    ],
    compiler_params=pltpu.CompilerParams(
        needs_layout_passes=False, use_tc_tiling_on_sc=False),
)(_seg_body)


BR = 1000
GRID = N // BR


def _norm_body(part_ref, out_ref):
    sums = jnp.sum(part_ref[...], axis=0, keepdims=True)
    out_ref[...] = jnp.where(sums > 0, lax.rsqrt(sums), 0.0)


def _norms(partial):
    bc = 2048
    grid = (2 * N + bc - 1) // bc
    return pl.pallas_call(
        _norm_body,
        grid=(grid,),
        in_specs=[pl.BlockSpec((NC * NS, bc), lambda i: (0, i))],
        out_specs=pl.BlockSpec((1, bc), lambda i: (0, i)),
        out_shape=jax.ShapeDtypeStruct((1, 2 * N), jnp.float32),
    )(partial)


def _init_body(h_ref, ns0_ref, ns1_ref, hr_ref, m0_ref, m1_ref):
    hr = jnp.maximum(h_ref[...], 0.0)
    hr_ref[...] = hr
    m0_ref[...] = hr * ns0_ref[...]
    m1_ref[...] = hr * ns1_ref[...]


def _init(h, ns0, ns1):
    spec = pl.BlockSpec((BR, D), lambda i: (i, 0))
    cspec = pl.BlockSpec((BR, 1), lambda i: (i, 0))
    shp = jax.ShapeDtypeStruct((N, D), jnp.float32)
    return pl.pallas_call(
        _init_body,
        grid=(GRID,),
        in_specs=[spec, cspec, cspec],
        out_specs=[spec, spec, spec],
        out_shape=[shp, shp, shp],
    )(h, ns0, ns1)


def _update_body(a0_ref, a1_ref, hr_ref, nd0_ref, nd1_ref, ns0_ref, ns1_ref,
                 m0_ref, m1_ref):
    hr = hr_ref[...]
    f0 = (1.0 - ALPHA) * a0_ref[...] * nd0_ref[...] + ALPHA * hr
    f1 = (1.0 - ALPHA) * a1_ref[...] * nd1_ref[...] + ALPHA * hr
    m0_ref[...] = f0 * ns0_ref[...]
    m1_ref[...] = f1 * ns1_ref[...]


def _update(a0, a1, hr, nd0, nd1, ns0, ns1):
    spec = pl.BlockSpec((BR, D), lambda i: (i, 0))
    cspec = pl.BlockSpec((BR, 1), lambda i: (i, 0))
    shp = jax.ShapeDtypeStruct((N, D), jnp.float32)
    return pl.pallas_call(
        _update_body,
        grid=(GRID,),
        in_specs=[spec, spec, spec, cspec, cspec, cspec, cspec],
        out_specs=[spec, spec],
        out_shape=[shp, shp],
    )(a0, a1, hr, nd0, nd1, ns0, ns1)


def _final_body(a0_ref, a1_ref, hr_ref, nd0_ref, nd1_ref, z0_ref, z1_ref):
    hr = hr_ref[...]
    z0_ref[...] = (1.0 - ALPHA) * a0_ref[...] * nd0_ref[...] + ALPHA * hr
    z1_ref[...] = (1.0 - ALPHA) * a1_ref[...] * nd1_ref[...] + ALPHA * hr


def _final(a0, a1, hr, nd0, nd1):
    spec = pl.BlockSpec((BR, D), lambda i: (i, 0))
    cspec = pl.BlockSpec((BR, 1), lambda i: (i, 0))
    shp = jax.ShapeDtypeStruct((N, D), jnp.float32)
    return pl.pallas_call(
        _final_body,
        grid=(GRID,),
        in_specs=[spec, spec, spec, cspec, cspec],
        out_specs=[spec, spec],
        out_shape=[shp, shp],
    )(a0, a1, hr, nd0, nd1)


def _fuse1_body(z0_ref, z1_ref, w1_ref, b1_ref, w2_ref, s_ref):
    @pl.when(pl.program_id(0) == 0)
    def _():
        s_ref[...] = jnp.zeros_like(s_ref)

    sums = []
    for z_ref in (z0_ref, z1_ref):
        t = jnp.tanh(
            jnp.dot(z_ref[...], w1_ref[...], preferred_element_type=jnp.float32)
            + b1_ref[...])
        v = jnp.dot(t, w2_ref[...], preferred_element_type=jnp.float32)
        sums.append(jnp.sum(v))
    s_ref[...] += jnp.reshape(jnp.stack(sums), (1, 2))


def _fuse1(z0, z1, W1, b1, W2):
    spec = pl.BlockSpec((BR, D), lambda i: (i, 0))
    return pl.pallas_call(
        _fuse1_body,
        grid=(GRID,),
        in_specs=[spec, spec,
                  pl.BlockSpec((D, HID), lambda i: (0, 0)),
                  pl.BlockSpec((1, HID), lambda i: (0, 0)),
                  pl.BlockSpec((HID, 1), lambda i: (0, 0))],
        out_specs=pl.BlockSpec((1, 2), lambda i: (0, 0)),
        out_shape=jax.ShapeDtypeStruct((1, 2), jnp.float32),
    )(z0, z1, W1, b1, W2)


def _fuse2_body(z0_ref, z1_ref, s_ref, out_ref):
    s = s_ref[...] * (1.0 / N)
    e = jnp.exp(s - jnp.max(s))
    b = e / jnp.sum(e)
    out_ref[...] = b[0:1, 0:1] * z0_ref[...] + b[0:1, 1:2] * z1_ref[...]


def _fuse2(z0, z1, s):
    spec = pl.BlockSpec((BR, D), lambda i: (i, 0))
    return pl.pallas_call(
        _fuse2_body,
        grid=(GRID,),
        in_specs=[spec, spec, pl.BlockSpec((1, 2), lambda i: (0, 0))],
        out_specs=spec,
        out_shape=jax.ShapeDtypeStruct((N, D), jnp.float32),
    )(z0, z1, s)


def kernel(h, edge_index0, edge_index1, W1, b1, W2):
    src0, dst0 = edge_index0[0], edge_index0[1]
    src1, dst1 = edge_index1[0], edge_index1[1]
    zrows = jnp.zeros((128, D), jnp.float32)

    part0 = _deg_kernel(src0, dst0)
    part1 = _deg_kernel(src1, dst1)
    norms0 = jnp.reshape(_norms(part0), (2, N, 1))
    norms1 = jnp.reshape(_norms(part1), (2, N, 1))
    ns0, nd0 = norms0[0], norms0[1]
    ns1, nd1 = norms1[0], norms1[1]

    hr, m0, m1 = _init(h, ns0, ns1)
    for k in range(K):
        a0 = _seg_kernel(m0, src0, dst0, zrows)
        a1 = _seg_kernel(m1, src1, dst1, zrows)
        if k < K - 1:
            m0, m1 = _update(a0, a1, hr, nd0, nd1, ns0, ns1)
        else:
            z0, z1 = _final(a0, a1, hr, nd0, nd1)

    s = _fuse1(z0, z1, W1, jnp.reshape(b1, (1, HID)), W2)
    return _fuse2(z0, z1, s)
